# baseline (speedup 1.0000x reference)
import functools

import jax
import jax.numpy as jnp
from jax.experimental import pallas as pl
from jax.experimental.pallas import tpu as pltpu

_DIM = 40
_K = 20
_ALPHA = 3.0
_ROWS_PER_BLOCK = 40
_ONE_BITS = 0x3F800000


def _nv_kernel(emb1_ref, w1_ref, b1_ref, emb2_ref, w2_ref, b2_ref,
               nv1_ref, nv2_ref):
    for m in range(4):
        x1 = jax.lax.dot_general(
            emb1_ref[m], w1_ref[m], (((1,), (1,)), ((), ())),
            preferred_element_type=jnp.float32)
        nv1_ref[m] = jnp.tanh(_ALPHA * (x1 + b1_ref[m][None, :]))
        x2 = jax.lax.dot_general(
            emb2_ref[m], w2_ref[m], (((1,), (1,)), ((), ())),
            preferred_element_type=jnp.float32)
        nv2_ref[m] = jnp.tanh(_ALPHA * (x2 + b2_ref[m][None, :]))


def _topk_kernel(nv1_ref, nv2_ref, out_ref, *, n_cols, k):
    x = nv1_ref[0]
    rows = x.shape[0]

    def _roll_prefix(e):
        col = jax.lax.broadcasted_iota(jnp.int32, e.shape, 1)
        prefix = e.astype(jnp.int32)
        shift = 1
        while shift < e.shape[1]:
            rolled = pltpu.roll(prefix, shift, 1)
            prefix = prefix + jnp.where(col >= shift, rolled, 0)
            shift *= 2
        return prefix

    def _generic():
        y = nv2_ref[0]
        a = jax.lax.dot_general(x, y, (((1,), (1,)), ((), ())),
                                preferred_element_type=jnp.float32)
        v = jnp.maximum(jnp.tanh(_ALPHA * a), 0.0)
        bits = jax.lax.bitcast_convert_type(v, jnp.int32)
        bits = jnp.maximum(bits, 0)

        def body(_, carry):
            lo, hi = carry
            mid = jax.lax.shift_right_logical(lo + hi, 1)
            cnt = jnp.sum((bits > mid).astype(jnp.int32), axis=1,
                          keepdims=True)
            big = cnt >= k
            lo = jnp.where(big, mid + 1, lo)
            hi = jnp.where(big, hi, mid)
            return lo, hi

        lo0 = jnp.zeros((rows, 1), jnp.int32)
        hi0 = jnp.full((rows, 1), _ONE_BITS, jnp.int32)
        _, t = jax.lax.fori_loop(0, 30, body, (lo0, hi0))
        gt = bits > t
        eq = bits == t
        need = k - jnp.sum(gt.astype(jnp.int32), axis=1, keepdims=True)
        sel_eq = eq & (_roll_prefix(eq) <= need)
        out_ref[...] = jnp.where(gt | sel_eq, v, 0.0)

    w = min(1024, n_cols)
    y_w = nv2_ref[0, :w, :]
    a_w = jax.lax.dot_general(x, y_w, (((1,), (1,)), ((), ())),
                              preferred_element_type=jnp.float32)
    eq1 = jnp.tanh(_ALPHA * a_w) == 1.0
    cnt_ones_w = jnp.sum(eq1.astype(jnp.int32), axis=1, keepdims=True)

    def _saturated_narrow():
        p = _roll_prefix(eq1)
        sel_w = eq1 & (p <= k)
        out_ref[:, :w] = sel_w.astype(jnp.float32)
        if w < n_cols:
            out_ref[:, w:] = jnp.zeros((rows, n_cols - w), jnp.float32)

    jax.lax.cond(jnp.all(cnt_ones_w >= k), _saturated_narrow, _generic)


def kernel(idx, emb1, emb2, W1, b1, W2, b2):
    n_sub, dim = emb1.shape[1], emb1.shape[2]
    n = 2 * n_sub

    nv1, nv2 = pl.pallas_call(
        _nv_kernel,
        out_shape=(
            jax.ShapeDtypeStruct((4, n_sub, dim), jnp.float32),
            jax.ShapeDtypeStruct((4, n_sub, dim), jnp.float32),
        ),
    )(emb1, W1, b1, emb2, W2, b2)

    zeros = jnp.zeros((n_sub, dim), jnp.float32)
    nv1cat = jnp.stack([
        jnp.concatenate([nv1[0], nv1[1]], axis=1),
        jnp.concatenate([nv1[2], nv1[3]], axis=1),
    ])
    nv2cat = jnp.stack([
        jnp.concatenate([
            jnp.concatenate([nv2[0], zeros], axis=1),
            jnp.concatenate([zeros, nv2[1]], axis=1),
        ], axis=0),
        jnp.concatenate([
            jnp.concatenate([nv2[2], zeros], axis=1),
            jnp.concatenate([zeros, nv2[3]], axis=1),
        ], axis=0),
    ])

    rpb = _ROWS_PER_BLOCK
    blocks_per_half = n_sub // rpb
    grid = (2 * blocks_per_half,)

    out = pl.pallas_call(
        functools.partial(_topk_kernel, n_cols=n, k=_K),
        grid=grid,
        in_specs=[
            pl.BlockSpec((1, rpb, 2 * dim),
                         lambda g: (g // blocks_per_half, g % blocks_per_half, 0)),
            pl.BlockSpec((1, n, 2 * dim),
                         lambda g: (g // blocks_per_half, 0, 0)),
        ],
        out_specs=pl.BlockSpec((rpb, n), lambda g: (g, 0)),
        out_shape=jax.ShapeDtypeStruct((n, n), jnp.float32),
    )(nv1cat, nv2cat)
    return out


# speedup vs baseline: 1.3022x; 1.3022x over previous
import functools

import jax
import jax.numpy as jnp
from jax.experimental import pallas as pl
from jax.experimental.pallas import tpu as pltpu

_DIM = 40
_K = 20
_ALPHA = 3.0
_ROWS_PER_BLOCK = 200
_ONE_BITS = 0x3F800000


def _nv_kernel(emb1_ref, w1_ref, b1_ref, emb2_ref, w2_ref, b2_ref,
               nv1_ref, nv2_ref):
    for m in range(4):
        x1 = jax.lax.dot_general(
            emb1_ref[m], w1_ref[m], (((1,), (1,)), ((), ())),
            preferred_element_type=jnp.float32)
        nv1_ref[m] = jnp.tanh(_ALPHA * (x1 + b1_ref[m][None, :]))
        x2 = jax.lax.dot_general(
            emb2_ref[m], w2_ref[m], (((1,), (1,)), ((), ())),
            preferred_element_type=jnp.float32)
        nv2_ref[m] = jnp.tanh(_ALPHA * (x2 + b2_ref[m][None, :]))


def _topk_kernel(nv1_ref, nv2_ref, out_ref, *, n_cols, k):
    x = nv1_ref[0]
    rows = x.shape[0]

    def _roll_prefix(e):
        col = jax.lax.broadcasted_iota(jnp.int32, e.shape, 1)
        prefix = e.astype(jnp.int32)
        shift = 1
        while shift < e.shape[1]:
            rolled = pltpu.roll(prefix, shift, 1)
            prefix = prefix + jnp.where(col >= shift, rolled, 0)
            shift *= 2
        return prefix

    def _generic():
        y = nv2_ref[0]
        a = jax.lax.dot_general(x, y, (((1,), (1,)), ((), ())),
                                preferred_element_type=jnp.float32)
        v = jnp.maximum(jnp.tanh(_ALPHA * a), 0.0)
        bits = jax.lax.bitcast_convert_type(v, jnp.int32)
        bits = jnp.maximum(bits, 0)

        def body(_, carry):
            lo, hi = carry
            mid = jax.lax.shift_right_logical(lo + hi, 1)
            cnt = jnp.sum((bits > mid).astype(jnp.int32), axis=1,
                          keepdims=True)
            big = cnt >= k
            lo = jnp.where(big, mid + 1, lo)
            hi = jnp.where(big, hi, mid)
            return lo, hi

        lo0 = jnp.zeros((rows, 1), jnp.int32)
        hi0 = jnp.full((rows, 1), _ONE_BITS, jnp.int32)
        _, t = jax.lax.fori_loop(0, 30, body, (lo0, hi0))
        gt = bits > t
        eq = bits == t
        need = k - jnp.sum(gt.astype(jnp.int32), axis=1, keepdims=True)
        sel_eq = eq & (_roll_prefix(eq) <= need)
        out_ref[...] = jnp.where(gt | sel_eq, v, 0.0)

    w = min(1024, n_cols)
    y_w = nv2_ref[0, :w, :]
    a_w = jax.lax.dot_general(x, y_w, (((1,), (1,)), ((), ())),
                              preferred_element_type=jnp.float32)
    eq1 = jnp.tanh(_ALPHA * a_w) == 1.0
    cnt_ones_w = jnp.sum(eq1.astype(jnp.int32), axis=1, keepdims=True)

    p = _roll_prefix(eq1)
    sel_w = eq1 & (p <= k)
    out_ref[:, :w] = sel_w.astype(jnp.float32)
    if w < n_cols:
        out_ref[:, w:] = jnp.zeros((rows, n_cols - w), jnp.float32)

    def _noop():
        pass

    jax.lax.cond(jnp.all(cnt_ones_w >= k), _noop, _generic)


def kernel(idx, emb1, emb2, W1, b1, W2, b2):
    n_sub, dim = emb1.shape[1], emb1.shape[2]
    n = 2 * n_sub

    nv1, nv2 = pl.pallas_call(
        _nv_kernel,
        out_shape=(
            jax.ShapeDtypeStruct((4, n_sub, dim), jnp.float32),
            jax.ShapeDtypeStruct((4, n_sub, dim), jnp.float32),
        ),
    )(emb1, W1, b1, emb2, W2, b2)

    zeros = jnp.zeros((n_sub, dim), jnp.float32)
    nv1cat = jnp.stack([
        jnp.concatenate([nv1[0], nv1[1]], axis=1),
        jnp.concatenate([nv1[2], nv1[3]], axis=1),
    ])
    nv2cat = jnp.stack([
        jnp.concatenate([
            jnp.concatenate([nv2[0], zeros], axis=1),
            jnp.concatenate([zeros, nv2[1]], axis=1),
        ], axis=0),
        jnp.concatenate([
            jnp.concatenate([nv2[2], zeros], axis=1),
            jnp.concatenate([zeros, nv2[3]], axis=1),
        ], axis=0),
    ])

    rpb = _ROWS_PER_BLOCK
    blocks_per_half = n_sub // rpb
    grid = (2 * blocks_per_half,)

    out = pl.pallas_call(
        functools.partial(_topk_kernel, n_cols=n, k=_K),
        grid=grid,
        in_specs=[
            pl.BlockSpec((1, rpb, 2 * dim),
                         lambda g: (g // blocks_per_half, g % blocks_per_half, 0)),
            pl.BlockSpec((1, n, 2 * dim),
                         lambda g: (g // blocks_per_half, 0, 0)),
        ],
        out_specs=pl.BlockSpec((rpb, n), lambda g: (g, 0)),
        out_shape=jax.ShapeDtypeStruct((n, n), jnp.float32),
        compiler_params=pltpu.CompilerParams(
            vmem_limit_bytes=100 * 1024 * 1024),
    )(nv1cat, nv2cat)
    return out


# speedup vs baseline: 1.7300x; 1.3286x over previous
import functools

import jax
import jax.numpy as jnp
from jax.experimental import pallas as pl
from jax.experimental.pallas import tpu as pltpu

_DIM = 40
_K = 20
_ALPHA = 3.0
_ROWS_PER_BLOCK = 200
_ONE_BITS = 0x3F800000


def _nv_kernel(emb1_ref, w1_ref, b1_ref, emb2_ref, w2_ref, b2_ref,
               nv1_ref, nv2_ref):
    for m in range(4):
        x1 = jax.lax.dot_general(
            emb1_ref[m], w1_ref[m], (((1,), (1,)), ((), ())),
            preferred_element_type=jnp.float32)
        nv1_ref[m] = jnp.tanh(_ALPHA * (x1 + b1_ref[m][None, :]))
        x2 = jax.lax.dot_general(
            emb2_ref[m], w2_ref[m], (((1,), (1,)), ((), ())),
            preferred_element_type=jnp.float32)
        nv2_ref[m] = jnp.tanh(_ALPHA * (x2 + b2_ref[m][None, :]))


def _topk_kernel(nv1_ref, nv2_ref, tril_ref, out_ref, *, n_cols, k):
    x = nv1_ref[0]
    rows = x.shape[0]

    def _roll_prefix(e):
        col = jax.lax.broadcasted_iota(jnp.int32, e.shape, 1)
        prefix = e.astype(jnp.int32)
        shift = 1
        while shift < e.shape[1]:
            rolled = pltpu.roll(prefix, shift, 1)
            prefix = prefix + jnp.where(col >= shift, rolled, 0)
            shift *= 2
        return prefix

    def _generic():
        y = nv2_ref[0]
        a = jax.lax.dot_general(x, y, (((1,), (1,)), ((), ())),
                                preferred_element_type=jnp.float32)
        v = jnp.maximum(jnp.tanh(_ALPHA * a), 0.0)
        bits = jax.lax.bitcast_convert_type(v, jnp.int32)
        bits = jnp.maximum(bits, 0)

        def body(_, carry):
            lo, hi = carry
            mid = jax.lax.shift_right_logical(lo + hi, 1)
            cnt = jnp.sum((bits > mid).astype(jnp.int32), axis=1,
                          keepdims=True)
            big = cnt >= k
            lo = jnp.where(big, mid + 1, lo)
            hi = jnp.where(big, hi, mid)
            return lo, hi

        lo0 = jnp.zeros((rows, 1), jnp.int32)
        hi0 = jnp.full((rows, 1), _ONE_BITS, jnp.int32)
        _, t = jax.lax.fori_loop(0, 30, body, (lo0, hi0))
        gt = bits > t
        eq = bits == t
        need = k - jnp.sum(gt.astype(jnp.int32), axis=1, keepdims=True)
        sel_eq = eq & (_roll_prefix(eq) <= need)
        out_ref[...] = jnp.where(gt | sel_eq, v, 0.0)

    w = min(1024, n_cols)
    y_w = nv2_ref[0, :w, :]
    a_w = jax.lax.dot_general(x, y_w, (((1,), (1,)), ((), ())),
                              preferred_element_type=jnp.float32)
    eq1 = jnp.tanh(_ALPHA * a_w) == 1.0
    cnt_ones_w = jnp.sum(eq1.astype(jnp.int32), axis=1, keepdims=True)

    def _saturated_narrow():
        p = jax.lax.dot_general(
            eq1.astype(jnp.bfloat16), tril_ref[...],
            (((1,), (0,)), ((), ())),
            preferred_element_type=jnp.float32)
        sel_w = eq1 & (p <= float(k))
        out_ref[:, :w] = sel_w.astype(jnp.float32)
        if w < n_cols:
            out_ref[:, w:] = jnp.zeros((rows, n_cols - w), jnp.float32)

    jax.lax.cond(jnp.all(cnt_ones_w >= k), _saturated_narrow, _generic)


def kernel(idx, emb1, emb2, W1, b1, W2, b2):
    n_sub, dim = emb1.shape[1], emb1.shape[2]
    n = 2 * n_sub

    nv1, nv2 = pl.pallas_call(
        _nv_kernel,
        out_shape=(
            jax.ShapeDtypeStruct((4, n_sub, dim), jnp.float32),
            jax.ShapeDtypeStruct((4, n_sub, dim), jnp.float32),
        ),
    )(emb1, W1, b1, emb2, W2, b2)

    zeros = jnp.zeros((n_sub, dim), jnp.float32)
    nv1cat = jnp.stack([
        jnp.concatenate([nv1[0], nv1[1]], axis=1),
        jnp.concatenate([nv1[2], nv1[3]], axis=1),
    ])
    nv2cat = jnp.stack([
        jnp.concatenate([
            jnp.concatenate([nv2[0], zeros], axis=1),
            jnp.concatenate([zeros, nv2[1]], axis=1),
        ], axis=0),
        jnp.concatenate([
            jnp.concatenate([nv2[2], zeros], axis=1),
            jnp.concatenate([zeros, nv2[3]], axis=1),
        ], axis=0),
    ])

    rpb = _ROWS_PER_BLOCK
    blocks_per_half = n_sub // rpb
    grid = (2 * blocks_per_half,)

    w = min(1024, n)
    col_ge_row = (jnp.arange(w)[:, None] <= jnp.arange(w)[None, :])
    tril = col_ge_row.astype(jnp.bfloat16)

    out = pl.pallas_call(
        functools.partial(_topk_kernel, n_cols=n, k=_K),
        grid=grid,
        in_specs=[
            pl.BlockSpec((1, rpb, 2 * dim),
                         lambda g: (g // blocks_per_half, g % blocks_per_half, 0)),
            pl.BlockSpec((1, n, 2 * dim),
                         lambda g: (g // blocks_per_half, 0, 0)),
            pl.BlockSpec((w, w), lambda g: (0, 0)),
        ],
        out_specs=pl.BlockSpec((rpb, n), lambda g: (g, 0)),
        out_shape=jax.ShapeDtypeStruct((n, n), jnp.float32),
        compiler_params=pltpu.CompilerParams(
            vmem_limit_bytes=100 * 1024 * 1024),
    )(nv1cat, nv2cat, tril)
    return out
